# baseline (speedup 1.0000x reference)
import functools

import jax
import jax.numpy as jnp
from jax import lax
from jax.experimental import pallas as pl
from jax.experimental.pallas import tpu as pltpu
from jax.experimental.pallas import tpu_sc as plsc

N = 100000
NG = 32
H = 128
L = 4
GP = 256
M1, M2 = 256, 128
NC = 64

NW = 32
NCK = 4
NPAD = 100352
CH = NPAD // NCK
RPW = CH // NW
RC = 112
NCH = RPW // RC
FV = GP // 16

BLK = 1792
NB = CH // BLK

NEG_INF = float("-inf")
BIG = 3.0e38


def _encoder_body(
    pos_ref, W_ffm_ref, b_ffm_ref, W_nnx_ref, b_nnx_ref,
    s1_ref, b1_ref, l1W_ref, l1b_ref, l2W_ref, l2b_ref,
    gps_ref, gpb2_ref, gpW_ref, gpb_ref,
    g_ref, *, nvalid,
):
    i = pl.program_id(0)
    x = jnp.dot(pos_ref[...], W_ffm_ref[...],
                preferred_element_type=jnp.float32) + b_ffm_ref[...]

    t = jnp.maximum(
        jnp.dot(x, W_nnx_ref[...], preferred_element_type=jnp.float32)
        + b_nnx_ref[...], 0.0)
    t = jnp.maximum(
        jnp.dot(t, l1W_ref[0], preferred_element_type=jnp.float32)
        + l1b_ref[0:1, :], 0.0)
    x = x + jnp.dot(t, l2W_ref[0], preferred_element_type=jnp.float32) \
        + l2b_ref[0:1, :]

    for il in range(1, L):
        t = jnp.maximum(x * s1_ref[il:il + 1, :] + b1_ref[il:il + 1, :], 0.0)
        t = jnp.maximum(
            jnp.dot(t, l1W_ref[il], preferred_element_type=jnp.float32)
            + l1b_ref[il:il + 1, :], 0.0)
        x = x + jnp.dot(t, l2W_ref[il], preferred_element_type=jnp.float32) \
            + l2b_ref[il:il + 1, :]

    g = jnp.dot(jnp.maximum(x * gps_ref[...] + gpb2_ref[...], 0.0),
                gpW_ref[...], preferred_element_type=jnp.float32) \
        + gpb_ref[...]

    rows = i * BLK + jax.lax.broadcasted_iota(jnp.int32, (BLK, 1), 0)
    g_ref[...] = jnp.where(rows < nvalid, g, NEG_INF)


def _pool_body(g_hbm, b_hbm, out_hbm,
               gb0, gb1, bb0, bb1, acc,
               sg0, sg1, sb0, sb1):
    ci = lax.axis_index("c")
    si = lax.axis_index("s")
    wid = si * 2 + ci
    base = wid * RPW
    lane = lax.broadcasted_iota(jnp.int32, (16,), 0)
    neg = jnp.full((16,), NEG_INF, jnp.float32)

    def init_body(q, _):
        acc[pl.ds(q * 16, 16)] = neg
        return 0
    lax.fori_loop(0, NG * GP // 16, init_body, 0)

    def start(ch, gb, bb, sg, sb):
        pltpu.make_async_copy(
            g_hbm.at[pl.ds((base + ch * RC) * GP, RC * GP)], gb, sg).start()
        pltpu.make_async_copy(
            b_hbm.at[pl.ds(base + ch * RC, RC)], bb, sb).start()

    def wait(gb, bb, sg, sb):
        pltpu.make_async_copy(
            g_hbm.at[pl.ds(base * GP, RC * GP)], gb, sg).wait()
        pltpu.make_async_copy(
            b_hbm.at[pl.ds(base, RC)], bb, sb).wait()

    def flush(cur, scr):
        coff = jnp.maximum(cur, 0) * GP
        for j in range(FV):
            acc[pl.ds(coff + j * 16, 16)] = scr[j]

    def process(gb, bb, cur, scr):
        def group_body(u, carry):
            cur, scr = carry
            bv = bb[pl.ds(u * 16, 16)]
            b0 = bv[0]
            bL = bv[15]

            @pl.when(jnp.logical_or(b0 != cur, bL != cur))
            def _replay(cur=cur, scr=scr):
                flush(cur, scr)
                c, s = cur, list(scr)
                for k in range(16):
                    bs = bv[k]
                    df = jnp.full((16,), bs - c).astype(jnp.float32)
                    sf = jnp.sign(df)
                    pen = BIG * (1.0 - 2.0 * sf * sf)
                    boff = bs * GP
                    roff = (u * 16 + k) * GP
                    for j in range(FV):
                        v = jnp.maximum(gb[pl.ds(roff + j * 16, 16)],
                                        jnp.minimum(s[j], pen))
                        acc[pl.ds(boff + j * 16, 16)] = v
                        s[j] = v
                    c = bs
            c = cur
            s = list(scr)
            for k in range(16):
                bs = bv[k]
                df = jnp.full((16,), bs - c).astype(jnp.float32)
                sf = jnp.sign(df)
                pen = BIG * (1.0 - 2.0 * sf * sf)
                roff = (u * 16 + k) * GP
                for j in range(FV):
                    s[j] = jnp.maximum(gb[pl.ds(roff + j * 16, 16)],
                                       jnp.minimum(s[j], pen))
                c = bs
            return (c, tuple(s))
        return lax.fori_loop(0, RC // 16, group_body, (cur, scr))

    start(0, gb0, bb0, sg0, sb0)
    start(1, gb1, bb1, sg1, sb1)
    cur0 = jnp.int32(-1)
    scr0 = tuple(neg for _ in range(FV))

    def chunk_body(h, carry):
        cur, scr = carry
        n0 = 2 * h
        wait(gb0, bb0, sg0, sb0)
        cur, scr = process(gb0, bb0, cur, scr)

        @pl.when(n0 + 2 < NCH)
        def _():
            start(n0 + 2, gb0, bb0, sg0, sb0)

        wait(gb1, bb1, sg1, sb1)
        cur, scr = process(gb1, bb1, cur, scr)

        @pl.when(n0 + 3 < NCH)
        def _():
            start(n0 + 3, gb1, bb1, sg1, sb1)

        return (cur, scr)

    cur, scr = lax.fori_loop(0, NCH // 2, chunk_body, (cur0, scr0))
    if NCH % 2:
        wait(gb0, bb0, sg0, sb0)
        cur, scr = process(gb0, bb0, cur, scr)
    flush(cur, scr)
    pltpu.sync_copy(acc, out_hbm.at[wid])


def _combine_body(part_ref, rW1_ref, rb1_ref, rW2_ref, rb2_ref,
                  rW3_ref, rb3_ref, out_ref):
    p = part_ref[0]
    for w in range(1, NCK * NW):
        p = jnp.maximum(p, part_ref[w])
    h = jnp.maximum(
        jnp.dot(p, rW1_ref[...], preferred_element_type=jnp.float32)
        + rb1_ref[...], 0.0)
    h = jnp.maximum(
        jnp.dot(h, rW2_ref[...], preferred_element_type=jnp.float32)
        + rb2_ref[...], 0.0)
    out_ref[...] = jnp.dot(h, rW3_ref[...],
                           preferred_element_type=jnp.float32) + rb3_ref[...]


def kernel(pos, batch, W_ffm, b_ffm, W_nnx, b_nnx, bn1_g, bn1_b, lin1_W,
           lin1_b, bn2_g, bn2_b, lin2_W, lin2_b, rc, gp_bn_g, gp_bn_b,
           gp_W, gp_b, reg_W1, reg_b1, reg_W2, reg_b2, reg_W3, reg_b3):
    inv = 1.0 / jnp.sqrt(1.0 + 1e-5)
    s1 = bn1_g * inv
    s2 = bn2_g * inv
    W_nnx_f = W_nnx * s1[0][None, :]
    b_nnx_f = (b_nnx * s1[0] + bn1_b[0]).reshape(1, H)
    lin1_Wf = lin1_W * s2[:, None, :]
    lin1_bf = lin1_b * s2 + bn2_b
    lin2_Wf = lin2_W * rc[:, None, None]
    lin2_bf = lin2_b * rc[:, None]
    gps = (gp_bn_g * inv).reshape(1, H)
    gpb2 = gp_bn_b.reshape(1, H)

    pos_pad = jnp.pad(pos, ((0, NPAD - N), (0, 0)))
    batch_pad = jnp.pad(batch, (0, NPAD - N), constant_values=NG - 1)

    full = lambda shape: pl.BlockSpec(shape, lambda i: tuple(0 for _ in shape))

    mesh = plsc.VectorSubcoreMesh(core_axis_name="c", subcore_axis_name="s")
    pool = pl.kernel(
        _pool_body,
        out_type=jax.ShapeDtypeStruct((NW, NG * GP), jnp.float32),
        mesh=mesh,
        scratch_types=[
            pltpu.VMEM((RC * GP,), jnp.float32),
            pltpu.VMEM((RC * GP,), jnp.float32),
            pltpu.VMEM((RC,), jnp.int32),
            pltpu.VMEM((RC,), jnp.int32),
            pltpu.VMEM((NG * GP,), jnp.float32),
            pltpu.SemaphoreType.DMA,
            pltpu.SemaphoreType.DMA,
            pltpu.SemaphoreType.DMA,
            pltpu.SemaphoreType.DMA,
        ],
    )

    parts = []
    for c in range(NCK):
        r0 = c * CH
        nvalid = min(max(N - r0, 0), CH)
        g_c = pl.pallas_call(
            functools.partial(_encoder_body, nvalid=nvalid),
            grid=(NB,),
            in_specs=[
                pl.BlockSpec((BLK, 3), lambda i: (i, 0)),
                full((3, H)), full((1, H)),
                full((H, H)), full((1, H)),
                full((L, H)), full((L, H)),
                full((L, H, H)), full((L, H)),
                full((L, H, H)), full((L, H)),
                full((1, H)), full((1, H)), full((H, GP)), full((1, GP)),
            ],
            out_specs=pl.BlockSpec((BLK, GP), lambda i: (i, 0)),
            out_shape=jax.ShapeDtypeStruct((CH, GP), jnp.float32),
        )(
            pos_pad[r0:r0 + CH],
            W_ffm, b_ffm.reshape(1, H),
            W_nnx_f, b_nnx_f,
            s1, bn1_b,
            lin1_Wf, lin1_bf,
            lin2_Wf, lin2_bf,
            gps, gpb2, gp_W, gp_b.reshape(1, GP),
        )
        parts.append(pool(g_c.reshape(CH * GP), batch_pad[r0:r0 + CH]))

    partials = jnp.concatenate(parts, axis=0)

    out = pl.pallas_call(
        _combine_body,
        grid=(1,),
        in_specs=[
            full((NCK * NW, NG, GP)),
            full((GP, M1)), full((1, M1)),
            full((M1, M2)), full((1, M2)),
            full((M2, NC)), full((1, NC)),
        ],
        out_specs=pl.BlockSpec((NG, NC), lambda i: (0, 0)),
        out_shape=jax.ShapeDtypeStruct((NG, NC), jnp.float32),
    )(
        partials.reshape(NCK * NW, NG, GP),
        reg_W1, reg_b1.reshape(1, M1),
        reg_W2, reg_b2.reshape(1, M2),
        reg_W3, reg_b3.reshape(1, NC),
    )
    return out


# speedup vs baseline: 1.1868x; 1.1868x over previous
import functools

import jax
import jax.numpy as jnp
from jax import lax
from jax.experimental import pallas as pl
from jax.experimental.pallas import tpu as pltpu
from jax.experimental.pallas import tpu_sc as plsc

N = 100000
NG = 32
H = 128
L = 4
GP = 256
M1, M2 = 256, 128
NC = 64

NW = 32
RPW = 3136
NPAD = NW * RPW
RC = 112
NCH = RPW // RC
FV = GP // 16

BLK = 3584
NB = NPAD // BLK

NEG_INF = float("-inf")
BIG = 3.0e38


def _encoder_body(
    pos_ref, W_ffm_ref, b_ffm_ref, W_nnx_ref, b_nnx_ref,
    s1_ref, b1_ref, l1W_ref, l1b_ref, l2W_ref, l2b_ref,
    gps_ref, gpb2_ref, gpW_ref, gpb_ref,
    g_ref,
):
    i = pl.program_id(0)
    x = jnp.dot(pos_ref[...], W_ffm_ref[...],
                preferred_element_type=jnp.float32) + b_ffm_ref[...]

    t = jnp.maximum(
        jnp.dot(x, W_nnx_ref[...], preferred_element_type=jnp.float32)
        + b_nnx_ref[...], 0.0)
    t = jnp.maximum(
        jnp.dot(t, l1W_ref[0], preferred_element_type=jnp.float32)
        + l1b_ref[0:1, :], 0.0)
    x = x + jnp.dot(t, l2W_ref[0], preferred_element_type=jnp.float32) \
        + l2b_ref[0:1, :]

    for il in range(1, L):
        t = jnp.maximum(x * s1_ref[il:il + 1, :] + b1_ref[il:il + 1, :], 0.0)
        t = jnp.maximum(
            jnp.dot(t, l1W_ref[il], preferred_element_type=jnp.float32)
            + l1b_ref[il:il + 1, :], 0.0)
        x = x + jnp.dot(t, l2W_ref[il], preferred_element_type=jnp.float32) \
            + l2b_ref[il:il + 1, :]

    g = jnp.dot(jnp.maximum(x * gps_ref[...] + gpb2_ref[...], 0.0),
                gpW_ref[...], preferred_element_type=jnp.float32) \
        + gpb_ref[...]

    rows = i * BLK + jax.lax.broadcasted_iota(jnp.int32, (BLK, 1), 0)
    g_ref[...] = jnp.where(rows < N, g, NEG_INF)


def _pool_body(g_hbm, b_hbm, out_hbm,
               gb0, gb1, bb0, bb1, acc,
               sg0, sg1, sb0, sb1):
    ci = lax.axis_index("c")
    si = lax.axis_index("s")
    wid = si * 2 + ci
    base = wid * RPW
    lane = lax.broadcasted_iota(jnp.int32, (16,), 0)
    neg = jnp.full((16,), NEG_INF, jnp.float32)

    def init_body(q, _):
        acc[pl.ds(q * 16, 16)] = neg
        return 0
    lax.fori_loop(0, NG * GP // 16, init_body, 0)

    def start(ch, gb, bb, sg, sb):
        pltpu.make_async_copy(
            g_hbm.at[pl.ds((base + ch * RC) * GP, RC * GP)], gb, sg).start()
        pltpu.make_async_copy(
            b_hbm.at[pl.ds(base + ch * RC, RC)], bb, sb).start()

    def wait(gb, bb, sg, sb):
        pltpu.make_async_copy(
            g_hbm.at[pl.ds(base * GP, RC * GP)], gb, sg).wait()
        pltpu.make_async_copy(
            b_hbm.at[pl.ds(base, RC)], bb, sb).wait()

    def flush(cur, scr):
        coff = jnp.maximum(cur, 0) * GP
        for j in range(FV):
            acc[pl.ds(coff + j * 16, 16)] = scr[j]

    def process(gb, bb, cur, scr):
        def group_body(u, carry):
            cur, scr = carry
            bv = bb[pl.ds(u * 16, 16)]
            b0 = bv[0]
            bL = bv[15]

            @pl.when(jnp.logical_or(b0 != cur, bL != cur))
            def _replay(cur=cur, scr=scr):
                flush(cur, scr)
                c, s = cur, list(scr)
                for k in range(16):
                    bs = bv[k]
                    df = jnp.full((16,), bs - c).astype(jnp.float32)
                    sf = jnp.sign(df)
                    pen = BIG * (1.0 - 2.0 * sf * sf)
                    boff = bs * GP
                    roff = (u * 16 + k) * GP
                    for j in range(FV):
                        v = jnp.maximum(gb[pl.ds(roff + j * 16, 16)],
                                        jnp.minimum(s[j], pen))
                        acc[pl.ds(boff + j * 16, 16)] = v
                        s[j] = v
                    c = bs
            c = cur
            s = list(scr)
            for k in range(16):
                bs = bv[k]
                df = jnp.full((16,), bs - c).astype(jnp.float32)
                sf = jnp.sign(df)
                pen = BIG * (1.0 - 2.0 * sf * sf)
                roff = (u * 16 + k) * GP
                for j in range(FV):
                    s[j] = jnp.maximum(gb[pl.ds(roff + j * 16, 16)],
                                       jnp.minimum(s[j], pen))
                c = bs
            return (c, tuple(s))
        return lax.fori_loop(0, RC // 16, group_body, (cur, scr))

    start(0, gb0, bb0, sg0, sb0)
    start(1, gb1, bb1, sg1, sb1)
    cur0 = jnp.int32(-1)
    scr0 = tuple(neg for _ in range(FV))

    def chunk_body(h, carry):
        cur, scr = carry
        n0 = 2 * h
        wait(gb0, bb0, sg0, sb0)
        cur, scr = process(gb0, bb0, cur, scr)

        @pl.when(n0 + 2 < NCH)
        def _():
            start(n0 + 2, gb0, bb0, sg0, sb0)

        wait(gb1, bb1, sg1, sb1)
        cur, scr = process(gb1, bb1, cur, scr)

        @pl.when(n0 + 3 < NCH)
        def _():
            start(n0 + 3, gb1, bb1, sg1, sb1)

        return (cur, scr)

    cur, scr = lax.fori_loop(0, NCH // 2, chunk_body, (cur0, scr0))
    flush(cur, scr)
    pltpu.sync_copy(acc, out_hbm.at[wid])


def _combine_body(part_ref, rW1_ref, rb1_ref, rW2_ref, rb2_ref,
                  rW3_ref, rb3_ref, out_ref):
    p = part_ref[0]
    for w in range(1, NW):
        p = jnp.maximum(p, part_ref[w])
    h = jnp.maximum(
        jnp.dot(p, rW1_ref[...], preferred_element_type=jnp.float32)
        + rb1_ref[...], 0.0)
    h = jnp.maximum(
        jnp.dot(h, rW2_ref[...], preferred_element_type=jnp.float32)
        + rb2_ref[...], 0.0)
    out_ref[...] = jnp.dot(h, rW3_ref[...],
                           preferred_element_type=jnp.float32) + rb3_ref[...]


def kernel(pos, batch, W_ffm, b_ffm, W_nnx, b_nnx, bn1_g, bn1_b, lin1_W,
           lin1_b, bn2_g, bn2_b, lin2_W, lin2_b, rc, gp_bn_g, gp_bn_b,
           gp_W, gp_b, reg_W1, reg_b1, reg_W2, reg_b2, reg_W3, reg_b3):
    inv = 1.0 / jnp.sqrt(1.0 + 1e-5)
    s1 = bn1_g * inv
    s2 = bn2_g * inv
    W_nnx_f = W_nnx * s1[0][None, :]
    b_nnx_f = (b_nnx * s1[0] + bn1_b[0]).reshape(1, H)
    lin1_Wf = lin1_W * s2[:, None, :]
    lin1_bf = lin1_b * s2 + bn2_b
    lin2_Wf = lin2_W * rc[:, None, None]
    lin2_bf = lin2_b * rc[:, None]
    gps = (gp_bn_g * inv).reshape(1, H)
    gpb2 = gp_bn_b.reshape(1, H)

    pos_pad = jnp.pad(pos, ((0, NPAD - N), (0, 0)))
    batch_pad = jnp.pad(batch, (0, NPAD - N), constant_values=NG - 1)

    full = lambda shape: pl.BlockSpec(shape, lambda i: tuple(0 for _ in shape))

    g = pl.pallas_call(
        _encoder_body,
        grid=(NB,),
        in_specs=[
            pl.BlockSpec((BLK, 3), lambda i: (i, 0)),
            full((3, H)), full((1, H)),
            full((H, H)), full((1, H)),
            full((L, H)), full((L, H)),
            full((L, H, H)), full((L, H)),
            full((L, H, H)), full((L, H)),
            full((1, H)), full((1, H)), full((H, GP)), full((1, GP)),
        ],
        out_specs=pl.BlockSpec((BLK, GP), lambda i: (i, 0)),
        out_shape=jax.ShapeDtypeStruct((NPAD, GP), jnp.float32),
    )(
        pos_pad,
        W_ffm, b_ffm.reshape(1, H),
        W_nnx_f, b_nnx_f,
        s1, bn1_b,
        lin1_Wf, lin1_bf,
        lin2_Wf, lin2_bf,
        gps, gpb2, gp_W, gp_b.reshape(1, GP),
    )

    mesh = plsc.VectorSubcoreMesh(core_axis_name="c", subcore_axis_name="s")
    partials = pl.kernel(
        _pool_body,
        out_type=jax.ShapeDtypeStruct((NW, NG * GP), jnp.float32),
        mesh=mesh,
        scratch_types=[
            pltpu.VMEM((RC * GP,), jnp.float32),
            pltpu.VMEM((RC * GP,), jnp.float32),
            pltpu.VMEM((RC,), jnp.int32),
            pltpu.VMEM((RC,), jnp.int32),
            pltpu.VMEM((NG * GP,), jnp.float32),
            pltpu.SemaphoreType.DMA,
            pltpu.SemaphoreType.DMA,
            pltpu.SemaphoreType.DMA,
            pltpu.SemaphoreType.DMA,
        ],
    )(g.reshape(NPAD * GP), batch_pad)

    out = pl.pallas_call(
        _combine_body,
        grid=(1,),
        in_specs=[
            full((NW, NG, GP)),
            full((GP, M1)), full((1, M1)),
            full((M1, M2)), full((1, M2)),
            full((M2, NC)), full((1, NC)),
        ],
        out_specs=pl.BlockSpec((NG, NC), lambda i: (0, 0)),
        out_shape=jax.ShapeDtypeStruct((NG, NC), jnp.float32),
    )(
        partials.reshape(NW, NG, GP),
        reg_W1, reg_b1.reshape(1, M1),
        reg_W2, reg_b2.reshape(1, M2),
        reg_W3, reg_b3.reshape(1, NC),
    )
    return out


# speedup vs baseline: 1.5519x; 1.3076x over previous
import functools

import jax
import jax.numpy as jnp
from jax import lax
from jax.experimental import pallas as pl
from jax.experimental.pallas import tpu as pltpu
from jax.experimental.pallas import tpu_sc as plsc

N = 100000
NG = 32
H = 128
L = 4
GP = 256
M1, M2 = 256, 128
NC = 64

NW = 32
RPW = 3136
NPAD = NW * RPW
RC = 112
NCH = RPW // RC
FV = GP // 16

BLK = 3584
NB = NPAD // BLK

NEG_INF = float("-inf")
BIG = 3.0e38


def _encoder_body(
    pos_ref, W_ffm_ref, b_ffm_ref, W_nnx_ref, b_nnx_ref,
    s1_ref, b1_ref, l1W_ref, l1b_ref, l2W_ref, l2b_ref,
    gps_ref, gpb2_ref, gpW_ref, gpb_ref,
    g_ref,
):
    i = pl.program_id(0)
    x = jnp.dot(pos_ref[...], W_ffm_ref[...],
                preferred_element_type=jnp.float32) + b_ffm_ref[...]

    t = jnp.maximum(
        jnp.dot(x, W_nnx_ref[...], preferred_element_type=jnp.float32)
        + b_nnx_ref[...], 0.0)
    t = jnp.maximum(
        jnp.dot(t, l1W_ref[0], preferred_element_type=jnp.float32)
        + l1b_ref[0:1, :], 0.0)
    x = x + jnp.dot(t, l2W_ref[0], preferred_element_type=jnp.float32) \
        + l2b_ref[0:1, :]

    for il in range(1, L):
        t = jnp.maximum(x * s1_ref[il:il + 1, :] + b1_ref[il:il + 1, :], 0.0)
        t = jnp.maximum(
            jnp.dot(t, l1W_ref[il], preferred_element_type=jnp.float32)
            + l1b_ref[il:il + 1, :], 0.0)
        x = x + jnp.dot(t, l2W_ref[il], preferred_element_type=jnp.float32) \
            + l2b_ref[il:il + 1, :]

    g = jnp.dot(jnp.maximum(x * gps_ref[...] + gpb2_ref[...], 0.0),
                gpW_ref[...], preferred_element_type=jnp.float32) \
        + gpb_ref[...]

    rows = i * BLK + jax.lax.broadcasted_iota(jnp.int32, (BLK, 1), 0)
    g_ref[...] = jnp.where(rows < N, g, NEG_INF).reshape(BLK * GP)


def _pool_body(g_hbm, b_hbm, out_hbm,
               gb0, gb1, bb0, bb1, acc,
               sg0, sg1, sb0, sb1):
    ci = lax.axis_index("c")
    si = lax.axis_index("s")
    wid = si * 2 + ci
    base = wid * RPW
    lane = lax.broadcasted_iota(jnp.int32, (16,), 0)
    neg = jnp.full((16,), NEG_INF, jnp.float32)

    def init_body(q, _):
        acc[pl.ds(q * 16, 16)] = neg
        return 0
    lax.fori_loop(0, NG * GP // 16, init_body, 0)

    def start(ch, gb, bb, sg, sb):
        pltpu.make_async_copy(
            g_hbm.at[pl.ds((base + ch * RC) * GP, RC * GP)], gb, sg).start()
        pltpu.make_async_copy(
            b_hbm.at[pl.ds(base + ch * RC, RC)], bb, sb).start()

    def wait(gb, bb, sg, sb):
        pltpu.make_async_copy(
            g_hbm.at[pl.ds(base * GP, RC * GP)], gb, sg).wait()
        pltpu.make_async_copy(
            b_hbm.at[pl.ds(base, RC)], bb, sb).wait()

    def flush(cur, scr):
        coff = jnp.maximum(cur, 0) * GP
        for j in range(FV):
            acc[pl.ds(coff + j * 16, 16)] = scr[j]

    def process(gb, bb, cur, scr):
        def group_body(u, carry):
            cur, scr = carry
            bv = bb[pl.ds(u * 16, 16)]
            b0 = bv[0]
            bL = bv[15]

            @pl.when(jnp.logical_or(b0 != cur, bL != cur))
            def _replay(cur=cur, scr=scr):
                flush(cur, scr)
                c, s = cur, list(scr)
                for k in range(16):
                    bs = bv[k]
                    df = jnp.full((16,), bs - c).astype(jnp.float32)
                    sf = jnp.sign(df)
                    pen = BIG * (1.0 - 2.0 * sf * sf)
                    boff = bs * GP
                    roff = (u * 16 + k) * GP
                    for j in range(FV):
                        v = jnp.maximum(gb[pl.ds(roff + j * 16, 16)],
                                        jnp.minimum(s[j], pen))
                        acc[pl.ds(boff + j * 16, 16)] = v
                        s[j] = v
                    c = bs
            c = cur
            s = list(scr)
            for k in range(16):
                bs = bv[k]
                df = jnp.full((16,), bs - c).astype(jnp.float32)
                sf = jnp.sign(df)
                pen = BIG * (1.0 - 2.0 * sf * sf)
                roff = (u * 16 + k) * GP
                for j in range(FV):
                    s[j] = jnp.maximum(gb[pl.ds(roff + j * 16, 16)],
                                       jnp.minimum(s[j], pen))
                c = bs
            return (c, tuple(s))
        return lax.fori_loop(0, RC // 16, group_body, (cur, scr))

    start(0, gb0, bb0, sg0, sb0)
    start(1, gb1, bb1, sg1, sb1)
    cur0 = jnp.int32(-1)
    scr0 = tuple(neg for _ in range(FV))

    def chunk_body(h, carry):
        cur, scr = carry
        n0 = 2 * h
        wait(gb0, bb0, sg0, sb0)
        cur, scr = process(gb0, bb0, cur, scr)

        @pl.when(n0 + 2 < NCH)
        def _():
            start(n0 + 2, gb0, bb0, sg0, sb0)

        wait(gb1, bb1, sg1, sb1)
        cur, scr = process(gb1, bb1, cur, scr)

        @pl.when(n0 + 3 < NCH)
        def _():
            start(n0 + 3, gb1, bb1, sg1, sb1)

        return (cur, scr)

    cur, scr = lax.fori_loop(0, NCH // 2, chunk_body, (cur0, scr0))
    flush(cur, scr)
    pltpu.sync_copy(acc, out_hbm.at[wid])


def _combine_body(part_ref, rW1_ref, rb1_ref, rW2_ref, rb2_ref,
                  rW3_ref, rb3_ref, out_ref):
    p = part_ref[0]
    for w in range(1, NW):
        p = jnp.maximum(p, part_ref[w])
    h = jnp.maximum(
        jnp.dot(p, rW1_ref[...], preferred_element_type=jnp.float32)
        + rb1_ref[...], 0.0)
    h = jnp.maximum(
        jnp.dot(h, rW2_ref[...], preferred_element_type=jnp.float32)
        + rb2_ref[...], 0.0)
    out_ref[...] = jnp.dot(h, rW3_ref[...],
                           preferred_element_type=jnp.float32) + rb3_ref[...]


def kernel(pos, batch, W_ffm, b_ffm, W_nnx, b_nnx, bn1_g, bn1_b, lin1_W,
           lin1_b, bn2_g, bn2_b, lin2_W, lin2_b, rc, gp_bn_g, gp_bn_b,
           gp_W, gp_b, reg_W1, reg_b1, reg_W2, reg_b2, reg_W3, reg_b3):
    inv = 1.0 / jnp.sqrt(1.0 + 1e-5)
    s1 = bn1_g * inv
    s2 = bn2_g * inv
    W_nnx_f = W_nnx * s1[0][None, :]
    b_nnx_f = (b_nnx * s1[0] + bn1_b[0]).reshape(1, H)
    lin1_Wf = lin1_W * s2[:, None, :]
    lin1_bf = lin1_b * s2 + bn2_b
    lin2_Wf = lin2_W * rc[:, None, None]
    lin2_bf = lin2_b * rc[:, None]
    gps = (gp_bn_g * inv).reshape(1, H)
    gpb2 = gp_bn_b.reshape(1, H)

    pos_pad = jnp.pad(pos, ((0, NPAD - N), (0, 0)))
    batch_pad = jnp.pad(batch, (0, NPAD - N), constant_values=NG - 1)

    full = lambda shape: pl.BlockSpec(shape, lambda i: tuple(0 for _ in shape))

    g = pl.pallas_call(
        _encoder_body,
        grid=(NB,),
        in_specs=[
            pl.BlockSpec((BLK, 3), lambda i: (i, 0)),
            full((3, H)), full((1, H)),
            full((H, H)), full((1, H)),
            full((L, H)), full((L, H)),
            full((L, H, H)), full((L, H)),
            full((L, H, H)), full((L, H)),
            full((1, H)), full((1, H)), full((H, GP)), full((1, GP)),
        ],
        out_specs=pl.BlockSpec((BLK * GP,), lambda i: (i,)),
        out_shape=jax.ShapeDtypeStruct((NPAD * GP,), jnp.float32),
    )(
        pos_pad,
        W_ffm, b_ffm.reshape(1, H),
        W_nnx_f, b_nnx_f,
        s1, bn1_b,
        lin1_Wf, lin1_bf,
        lin2_Wf, lin2_bf,
        gps, gpb2, gp_W, gp_b.reshape(1, GP),
    )

    mesh = plsc.VectorSubcoreMesh(core_axis_name="c", subcore_axis_name="s")
    partials = pl.kernel(
        _pool_body,
        out_type=jax.ShapeDtypeStruct((NW, NG * GP), jnp.float32),
        mesh=mesh,
        scratch_types=[
            pltpu.VMEM((RC * GP,), jnp.float32),
            pltpu.VMEM((RC * GP,), jnp.float32),
            pltpu.VMEM((RC,), jnp.int32),
            pltpu.VMEM((RC,), jnp.int32),
            pltpu.VMEM((NG * GP,), jnp.float32),
            pltpu.SemaphoreType.DMA,
            pltpu.SemaphoreType.DMA,
            pltpu.SemaphoreType.DMA,
            pltpu.SemaphoreType.DMA,
        ],
    )(g, batch_pad)

    out = pl.pallas_call(
        _combine_body,
        grid=(1,),
        in_specs=[
            full((NW, NG, GP)),
            full((GP, M1)), full((1, M1)),
            full((M1, M2)), full((1, M2)),
            full((M2, NC)), full((1, NC)),
        ],
        out_specs=pl.BlockSpec((NG, NC), lambda i: (0, 0)),
        out_shape=jax.ShapeDtypeStruct((NG, NC), jnp.float32),
    )(
        partials.reshape(NW, NG, GP),
        reg_W1, reg_b1.reshape(1, M1),
        reg_W2, reg_b2.reshape(1, M2),
        reg_W3, reg_b3.reshape(1, NC),
    )
    return out


# speedup vs baseline: 1.5580x; 1.0040x over previous
import functools

import jax
import jax.numpy as jnp
from jax import lax
from jax.experimental import pallas as pl
from jax.experimental.pallas import tpu as pltpu
from jax.experimental.pallas import tpu_sc as plsc

N = 100000
NG = 32
H = 128
L = 4
GP = 256
M1, M2 = 256, 128
NC = 64

NW = 32
RPW = 3136
NPAD = NW * RPW
RC = 224
NCH = RPW // RC
FV = GP // 16

BLK = 3584
NB = NPAD // BLK

NEG_INF = float("-inf")
BIG = 3.0e38


def _encoder_body(
    pos_ref, W_ffm_ref, b_ffm_ref, W_nnx_ref, b_nnx_ref,
    s1_ref, b1_ref, l1W_ref, l1b_ref, l2W_ref, l2b_ref,
    gps_ref, gpb2_ref, gpW_ref, gpb_ref,
    g_ref,
):
    i = pl.program_id(0)
    x = jnp.dot(pos_ref[...], W_ffm_ref[...],
                preferred_element_type=jnp.float32) + b_ffm_ref[...]

    t = jnp.maximum(
        jnp.dot(x, W_nnx_ref[...], preferred_element_type=jnp.float32)
        + b_nnx_ref[...], 0.0)
    t = jnp.maximum(
        jnp.dot(t, l1W_ref[0], preferred_element_type=jnp.float32)
        + l1b_ref[0:1, :], 0.0)
    x = x + jnp.dot(t, l2W_ref[0], preferred_element_type=jnp.float32) \
        + l2b_ref[0:1, :]

    for il in range(1, L):
        t = jnp.maximum(x * s1_ref[il:il + 1, :] + b1_ref[il:il + 1, :], 0.0)
        t = jnp.maximum(
            jnp.dot(t, l1W_ref[il], preferred_element_type=jnp.float32)
            + l1b_ref[il:il + 1, :], 0.0)
        x = x + jnp.dot(t, l2W_ref[il], preferred_element_type=jnp.float32) \
            + l2b_ref[il:il + 1, :]

    g = jnp.dot(jnp.maximum(x * gps_ref[...] + gpb2_ref[...], 0.0),
                gpW_ref[...], preferred_element_type=jnp.float32) \
        + gpb_ref[...]

    rows = i * BLK + jax.lax.broadcasted_iota(jnp.int32, (BLK, 1), 0)
    g_ref[...] = jnp.where(rows < N, g, NEG_INF).reshape(BLK * GP)


def _pool_body(g_hbm, b_hbm, out_hbm,
               gb0, gb1, bb0, bb1, acc,
               sg0, sg1, sb0, sb1):
    ci = lax.axis_index("c")
    si = lax.axis_index("s")
    wid = si * 2 + ci
    base = wid * RPW
    lane = lax.broadcasted_iota(jnp.int32, (16,), 0)
    neg = jnp.full((16,), NEG_INF, jnp.float32)

    def init_body(q, _):
        acc[pl.ds(q * 16, 16)] = neg
        return 0
    lax.fori_loop(0, NG * GP // 16, init_body, 0)

    def start(ch, gb, bb, sg, sb):
        pltpu.make_async_copy(
            g_hbm.at[pl.ds((base + ch * RC) * GP, RC * GP)], gb, sg).start()
        pltpu.make_async_copy(
            b_hbm.at[pl.ds(base + ch * RC, RC)], bb, sb).start()

    def wait(gb, bb, sg, sb):
        pltpu.make_async_copy(
            g_hbm.at[pl.ds(base * GP, RC * GP)], gb, sg).wait()
        pltpu.make_async_copy(
            b_hbm.at[pl.ds(base, RC)], bb, sb).wait()

    def flush(cur, scr):
        coff = jnp.maximum(cur, 0) * GP
        for j in range(FV):
            acc[pl.ds(coff + j * 16, 16)] = scr[j]

    def process(gb, bb, cur, scr):
        def group_body(u, carry):
            cur, scr = carry
            bv = bb[pl.ds(u * 16, 16)]
            b0 = bv[0]
            bL = bv[15]

            @pl.when(jnp.logical_or(b0 != cur, bL != cur))
            def _replay(cur=cur, scr=scr):
                flush(cur, scr)
                c, s = cur, list(scr)
                for k in range(16):
                    bs = bv[k]
                    df = jnp.full((16,), bs - c).astype(jnp.float32)
                    sf = jnp.sign(df)
                    pen = BIG * (1.0 - 2.0 * sf * sf)
                    boff = bs * GP
                    roff = (u * 16 + k) * GP
                    for j in range(FV):
                        v = jnp.maximum(gb[pl.ds(roff + j * 16, 16)],
                                        jnp.minimum(s[j], pen))
                        acc[pl.ds(boff + j * 16, 16)] = v
                        s[j] = v
                    c = bs
            c = cur
            s = list(scr)
            for k in range(16):
                bs = bv[k]
                df = jnp.full((16,), bs - c).astype(jnp.float32)
                sf = jnp.sign(df)
                pen = BIG * (1.0 - 2.0 * sf * sf)
                roff = (u * 16 + k) * GP
                for j in range(FV):
                    s[j] = jnp.maximum(gb[pl.ds(roff + j * 16, 16)],
                                       jnp.minimum(s[j], pen))
                c = bs
            return (c, tuple(s))
        return lax.fori_loop(0, RC // 16, group_body, (cur, scr))

    start(0, gb0, bb0, sg0, sb0)
    start(1, gb1, bb1, sg1, sb1)
    cur0 = jnp.int32(-1)
    scr0 = tuple(neg for _ in range(FV))

    def chunk_body(h, carry):
        cur, scr = carry
        n0 = 2 * h
        wait(gb0, bb0, sg0, sb0)
        cur, scr = process(gb0, bb0, cur, scr)

        @pl.when(n0 + 2 < NCH)
        def _():
            start(n0 + 2, gb0, bb0, sg0, sb0)

        wait(gb1, bb1, sg1, sb1)
        cur, scr = process(gb1, bb1, cur, scr)

        @pl.when(n0 + 3 < NCH)
        def _():
            start(n0 + 3, gb1, bb1, sg1, sb1)

        return (cur, scr)

    cur, scr = lax.fori_loop(0, NCH // 2, chunk_body, (cur0, scr0))
    flush(cur, scr)
    pltpu.sync_copy(acc, out_hbm.at[wid])


def _combine_body(part_ref, rW1_ref, rb1_ref, rW2_ref, rb2_ref,
                  rW3_ref, rb3_ref, out_ref):
    p = part_ref[0]
    for w in range(1, NW):
        p = jnp.maximum(p, part_ref[w])
    h = jnp.maximum(
        jnp.dot(p, rW1_ref[...], preferred_element_type=jnp.float32)
        + rb1_ref[...], 0.0)
    h = jnp.maximum(
        jnp.dot(h, rW2_ref[...], preferred_element_type=jnp.float32)
        + rb2_ref[...], 0.0)
    out_ref[...] = jnp.dot(h, rW3_ref[...],
                           preferred_element_type=jnp.float32) + rb3_ref[...]


def kernel(pos, batch, W_ffm, b_ffm, W_nnx, b_nnx, bn1_g, bn1_b, lin1_W,
           lin1_b, bn2_g, bn2_b, lin2_W, lin2_b, rc, gp_bn_g, gp_bn_b,
           gp_W, gp_b, reg_W1, reg_b1, reg_W2, reg_b2, reg_W3, reg_b3):
    inv = 1.0 / jnp.sqrt(1.0 + 1e-5)
    s1 = bn1_g * inv
    s2 = bn2_g * inv
    W_nnx_f = W_nnx * s1[0][None, :]
    b_nnx_f = (b_nnx * s1[0] + bn1_b[0]).reshape(1, H)
    lin1_Wf = lin1_W * s2[:, None, :]
    lin1_bf = lin1_b * s2 + bn2_b
    lin2_Wf = lin2_W * rc[:, None, None]
    lin2_bf = lin2_b * rc[:, None]
    gps = (gp_bn_g * inv).reshape(1, H)
    gpb2 = gp_bn_b.reshape(1, H)

    pos_pad = jnp.pad(pos, ((0, NPAD - N), (0, 0)))
    batch_pad = jnp.pad(batch, (0, NPAD - N), constant_values=NG - 1)

    full = lambda shape: pl.BlockSpec(shape, lambda i: tuple(0 for _ in shape))

    g = pl.pallas_call(
        _encoder_body,
        grid=(NB,),
        in_specs=[
            pl.BlockSpec((BLK, 3), lambda i: (i, 0)),
            full((3, H)), full((1, H)),
            full((H, H)), full((1, H)),
            full((L, H)), full((L, H)),
            full((L, H, H)), full((L, H)),
            full((L, H, H)), full((L, H)),
            full((1, H)), full((1, H)), full((H, GP)), full((1, GP)),
        ],
        out_specs=pl.BlockSpec((BLK * GP,), lambda i: (i,)),
        out_shape=jax.ShapeDtypeStruct((NPAD * GP,), jnp.float32),
    )(
        pos_pad,
        W_ffm, b_ffm.reshape(1, H),
        W_nnx_f, b_nnx_f,
        s1, bn1_b,
        lin1_Wf, lin1_bf,
        lin2_Wf, lin2_bf,
        gps, gpb2, gp_W, gp_b.reshape(1, GP),
    )

    mesh = plsc.VectorSubcoreMesh(core_axis_name="c", subcore_axis_name="s")
    partials = pl.kernel(
        _pool_body,
        out_type=jax.ShapeDtypeStruct((NW, NG * GP), jnp.float32),
        mesh=mesh,
        scratch_types=[
            pltpu.VMEM((RC * GP,), jnp.float32),
            pltpu.VMEM((RC * GP,), jnp.float32),
            pltpu.VMEM((RC,), jnp.int32),
            pltpu.VMEM((RC,), jnp.int32),
            pltpu.VMEM((NG * GP,), jnp.float32),
            pltpu.SemaphoreType.DMA,
            pltpu.SemaphoreType.DMA,
            pltpu.SemaphoreType.DMA,
            pltpu.SemaphoreType.DMA,
        ],
    )(g, batch_pad)

    out = pl.pallas_call(
        _combine_body,
        grid=(1,),
        in_specs=[
            full((NW, NG, GP)),
            full((GP, M1)), full((1, M1)),
            full((M1, M2)), full((1, M2)),
            full((M2, NC)), full((1, NC)),
        ],
        out_specs=pl.BlockSpec((NG, NC), lambda i: (0, 0)),
        out_shape=jax.ShapeDtypeStruct((NG, NC), jnp.float32),
    )(
        partials.reshape(NW, NG, GP),
        reg_W1, reg_b1.reshape(1, M1),
        reg_W2, reg_b2.reshape(1, M2),
        reg_W3, reg_b3.reshape(1, NC),
    )
    return out


# speedup vs baseline: 1.5585x; 1.0003x over previous
import functools

import jax
import jax.numpy as jnp
from jax import lax
from jax.experimental import pallas as pl
from jax.experimental.pallas import tpu as pltpu
from jax.experimental.pallas import tpu_sc as plsc

N = 100000
NG = 32
H = 128
L = 4
GP = 256
M1, M2 = 256, 128
NC = 64

NW = 32
RPW = 3136
NPAD = NW * RPW
RC = 224
NCH = RPW // RC
FV = GP // 16

BLK = 3584
NB = NPAD // BLK

NEG_INF = float("-inf")
BIG = 3.0e38


def _encoder_body(
    pos_ref, W_ffm_ref, b_ffm_ref, W_nnx_ref, b_nnx_ref,
    s1_ref, b1_ref, l1W_ref, l1b_ref, l2W_ref, l2b_ref,
    gps_ref, gpb2_ref, gpW_ref, gpb_ref,
    g_ref,
):
    i = pl.program_id(0)
    x = jnp.dot(pos_ref[...], W_ffm_ref[...],
                preferred_element_type=jnp.float32) + b_ffm_ref[...]

    t = jnp.maximum(
        jnp.dot(x, W_nnx_ref[...], preferred_element_type=jnp.float32)
        + b_nnx_ref[...], 0.0)
    t = jnp.maximum(
        jnp.dot(t, l1W_ref[0], preferred_element_type=jnp.float32)
        + l1b_ref[0:1, :], 0.0)
    x = x + jnp.dot(t, l2W_ref[0], preferred_element_type=jnp.float32) \
        + l2b_ref[0:1, :]

    for il in range(1, L):
        t = jnp.maximum(x * s1_ref[il:il + 1, :] + b1_ref[il:il + 1, :], 0.0)
        t = jnp.maximum(
            jnp.dot(t, l1W_ref[il], preferred_element_type=jnp.float32)
            + l1b_ref[il:il + 1, :], 0.0)
        x = x + jnp.dot(t, l2W_ref[il], preferred_element_type=jnp.float32) \
            + l2b_ref[il:il + 1, :]

    g = jnp.dot(jnp.maximum(x * gps_ref[...] + gpb2_ref[...], 0.0),
                gpW_ref[...], preferred_element_type=jnp.float32) \
        + gpb_ref[...]

    rows = i * BLK + jax.lax.broadcasted_iota(jnp.int32, (BLK, 1), 0)
    g_ref[...] = jnp.where(rows < N, g, NEG_INF).reshape(BLK * GP)


def _pool_body(g_hbm, b_hbm, out_hbm,
               gb0, gb1, bb0, bb1, acc,
               sg0, sg1, sb0, sb1):
    ci = lax.axis_index("c")
    si = lax.axis_index("s")
    wid = si * 2 + ci
    base = wid * RPW
    neg = jnp.full((16,), NEG_INF, jnp.float32)

    def init_body(q, _):
        acc[pl.ds(q * 16, 16)] = neg
        return 0
    lax.fori_loop(0, NG * GP // 16, init_body, 0)

    def start(ch, gb, bb, sg, sb):
        pltpu.make_async_copy(
            g_hbm.at[pl.ds((base + ch * RC) * GP, RC * GP)], gb, sg).start()
        pltpu.make_async_copy(
            b_hbm.at[pl.ds(base + ch * RC, RC)], bb, sb).start()

    def wait(gb, bb, sg, sb):
        pltpu.make_async_copy(
            g_hbm.at[pl.ds(base * GP, RC * GP)], gb, sg).wait()
        pltpu.make_async_copy(
            b_hbm.at[pl.ds(base, RC)], bb, sb).wait()

    def flush(cur, scr):
        coff = jnp.maximum(cur, 0) * GP
        for j in range(FV):
            acc[pl.ds(coff + j * 16, 16)] = scr[j]

    def process(gb, bb, cur, scr):
        def group_body(u, carry):
            cur, scr = carry
            bv = bb[pl.ds(u * 16, 16)]
            b0 = bv[0]
            bL = bv[15]

            @pl.when(jnp.logical_or(b0 != cur, bL != cur))
            def _replay(cur=cur, scr=scr):
                flush(cur, scr)
                c, s = cur, list(scr)
                for k in range(16):
                    bs = bv[k]
                    df = jnp.full((16,), bs - c).astype(jnp.float32)
                    sf = jnp.sign(df)
                    pen = BIG * (1.0 - 2.0 * sf * sf)
                    boff = bs * GP
                    roff = (u * 16 + k) * GP
                    for j in range(FV):
                        v = jnp.maximum(gb[pl.ds(roff + j * 16, 16)],
                                        jnp.minimum(s[j], pen))
                        acc[pl.ds(boff + j * 16, 16)] = v
                        s[j] = v
                    c = bs
            c = cur
            s = list(scr)
            for k in range(16):
                bs = bv[k]
                df = jnp.full((16,), bs - c).astype(jnp.float32)
                sf = jnp.sign(df)
                pen = BIG * (1.0 - 2.0 * sf * sf)
                roff = (u * 16 + k) * GP
                for j in range(FV):
                    s[j] = jnp.maximum(gb[pl.ds(roff + j * 16, 16)],
                                       jnp.minimum(s[j], pen))
                c = bs
            return (c, tuple(s))
        return lax.fori_loop(0, RC // 16, group_body, (cur, scr))

    start(0, gb0, bb0, sg0, sb0)
    start(1, gb1, bb1, sg1, sb1)
    cur0 = jnp.int32(-1)
    scr0 = tuple(neg for _ in range(FV))

    def chunk_body(h, carry):
        cur, scr = carry
        n0 = 2 * h
        wait(gb0, bb0, sg0, sb0)
        cur, scr = process(gb0, bb0, cur, scr)

        @pl.when(n0 + 2 < NCH)
        def _():
            start(n0 + 2, gb0, bb0, sg0, sb0)

        wait(gb1, bb1, sg1, sb1)
        cur, scr = process(gb1, bb1, cur, scr)

        @pl.when(n0 + 3 < NCH)
        def _():
            start(n0 + 3, gb1, bb1, sg1, sb1)

        return (cur, scr)

    cur, scr = lax.fori_loop(0, NCH // 2, chunk_body, (cur0, scr0))
    flush(cur, scr)
    pltpu.sync_copy(acc, out_hbm.at[wid])


def _combine_body(part_ref, rW1_ref, rb1_ref, rW2_ref, rb2_ref,
                  rW3_ref, rb3_ref, out_ref):
    p = part_ref[0]
    for w in range(1, NW):
        p = jnp.maximum(p, part_ref[w])
    h = jnp.maximum(
        jnp.dot(p, rW1_ref[...], preferred_element_type=jnp.float32)
        + rb1_ref[...], 0.0)
    h = jnp.maximum(
        jnp.dot(h, rW2_ref[...], preferred_element_type=jnp.float32)
        + rb2_ref[...], 0.0)
    out_ref[...] = jnp.dot(h, rW3_ref[...],
                           preferred_element_type=jnp.float32) + rb3_ref[...]


def kernel(pos, batch, W_ffm, b_ffm, W_nnx, b_nnx, bn1_g, bn1_b, lin1_W,
           lin1_b, bn2_g, bn2_b, lin2_W, lin2_b, rc, gp_bn_g, gp_bn_b,
           gp_W, gp_b, reg_W1, reg_b1, reg_W2, reg_b2, reg_W3, reg_b3):
    inv = 1.0 / jnp.sqrt(1.0 + 1e-5)
    s1 = bn1_g * inv
    s2 = bn2_g * inv
    W_nnx_f = W_nnx * s1[0][None, :]
    b_nnx_f = (b_nnx * s1[0] + bn1_b[0]).reshape(1, H)
    lin1_Wf = lin1_W * s2[:, None, :]
    lin1_bf = lin1_b * s2 + bn2_b
    lin2_Wf = lin2_W * rc[:, None, None]
    lin2_bf = lin2_b * rc[:, None]
    gps = (gp_bn_g * inv).reshape(1, H)
    gpb2 = gp_bn_b.reshape(1, H)

    pos_pad = jnp.pad(pos, ((0, NPAD - N), (0, 0)))
    batch_pad = jnp.pad(batch, (0, NPAD - N), constant_values=NG - 1)

    full = lambda shape: pl.BlockSpec(shape, lambda i: tuple(0 for _ in shape))

    g = pl.pallas_call(
        _encoder_body,
        grid=(NB,),
        in_specs=[
            pl.BlockSpec((BLK, 3), lambda i: (i, 0)),
            full((3, H)), full((1, H)),
            full((H, H)), full((1, H)),
            full((L, H)), full((L, H)),
            full((L, H, H)), full((L, H)),
            full((L, H, H)), full((L, H)),
            full((1, H)), full((1, H)), full((H, GP)), full((1, GP)),
        ],
        out_specs=pl.BlockSpec((BLK * GP,), lambda i: (i,)),
        out_shape=jax.ShapeDtypeStruct((NPAD * GP,), jnp.float32),
    )(
        pos_pad,
        W_ffm, b_ffm.reshape(1, H),
        W_nnx_f, b_nnx_f,
        s1, bn1_b,
        lin1_Wf, lin1_bf,
        lin2_Wf, lin2_bf,
        gps, gpb2, gp_W, gp_b.reshape(1, GP),
    )

    mesh = plsc.VectorSubcoreMesh(core_axis_name="c", subcore_axis_name="s")
    partials = pl.kernel(
        _pool_body,
        out_type=jax.ShapeDtypeStruct((NW, NG * GP), jnp.float32),
        mesh=mesh,
        scratch_types=[
            pltpu.VMEM((RC * GP,), jnp.float32),
            pltpu.VMEM((RC * GP,), jnp.float32),
            pltpu.VMEM((RC,), jnp.int32),
            pltpu.VMEM((RC,), jnp.int32),
            pltpu.VMEM((NG * GP,), jnp.float32),
            pltpu.SemaphoreType.DMA,
            pltpu.SemaphoreType.DMA,
            pltpu.SemaphoreType.DMA,
            pltpu.SemaphoreType.DMA,
        ],
    )(g, batch_pad)

    out = pl.pallas_call(
        _combine_body,
        grid=(1,),
        in_specs=[
            full((NW, NG, GP)),
            full((GP, M1)), full((1, M1)),
            full((M1, M2)), full((1, M2)),
            full((M2, NC)), full((1, NC)),
        ],
        out_specs=pl.BlockSpec((NG, NC), lambda i: (0, 0)),
        out_shape=jax.ShapeDtypeStruct((NG, NC), jnp.float32),
    )(
        partials.reshape(NW, NG, GP),
        reg_W1, reg_b1.reshape(1, M1),
        reg_W2, reg_b2.reshape(1, M2),
        reg_W3, reg_b3.reshape(1, NC),
    )
    return out


# speedup vs baseline: 1.5648x; 1.0040x over previous
import functools

import jax
import jax.numpy as jnp
from jax import lax
from jax.experimental import pallas as pl
from jax.experimental.pallas import tpu as pltpu
from jax.experimental.pallas import tpu_sc as plsc

N = 100000
NG = 32
H = 128
L = 4
GP = 256
M1, M2 = 256, 128
NC = 64

NW = 32
RPW = 3136
NPAD = NW * RPW
RC = 224
NCH = RPW // RC
FV = GP // 16

BLK = 7168
NB = NPAD // BLK

NEG_INF = float("-inf")
BIG = 3.0e38


def _encoder_body(
    pos_ref, W_ffm_ref, b_ffm_ref, W_nnx_ref, b_nnx_ref,
    s1_ref, b1_ref, l1W_ref, l1b_ref, l2W_ref, l2b_ref,
    gps_ref, gpb2_ref, gpW_ref, gpb_ref,
    g_ref,
):
    i = pl.program_id(0)
    x = jnp.dot(pos_ref[...], W_ffm_ref[...],
                preferred_element_type=jnp.float32) + b_ffm_ref[...]

    t = jnp.maximum(
        jnp.dot(x, W_nnx_ref[...], preferred_element_type=jnp.float32)
        + b_nnx_ref[...], 0.0)
    t = jnp.maximum(
        jnp.dot(t, l1W_ref[0], preferred_element_type=jnp.float32)
        + l1b_ref[0:1, :], 0.0)
    x = x + jnp.dot(t, l2W_ref[0], preferred_element_type=jnp.float32) \
        + l2b_ref[0:1, :]

    for il in range(1, L):
        t = jnp.maximum(x * s1_ref[il:il + 1, :] + b1_ref[il:il + 1, :], 0.0)
        t = jnp.maximum(
            jnp.dot(t, l1W_ref[il], preferred_element_type=jnp.float32)
            + l1b_ref[il:il + 1, :], 0.0)
        x = x + jnp.dot(t, l2W_ref[il], preferred_element_type=jnp.float32) \
            + l2b_ref[il:il + 1, :]

    g = jnp.dot(jnp.maximum(x * gps_ref[...] + gpb2_ref[...], 0.0),
                gpW_ref[...], preferred_element_type=jnp.float32) \
        + gpb_ref[...]

    rows = i * BLK + jax.lax.broadcasted_iota(jnp.int32, (BLK, 1), 0)
    g_ref[...] = jnp.where(rows < N, g, NEG_INF).reshape(BLK * GP)


def _pool_body(g_hbm, b_hbm, out_hbm,
               gb0, gb1, bb0, bb1, acc,
               sg0, sg1, sb0, sb1):
    ci = lax.axis_index("c")
    si = lax.axis_index("s")
    wid = si * 2 + ci
    base = wid * RPW
    neg = jnp.full((16,), NEG_INF, jnp.float32)

    def init_body(q, _):
        acc[pl.ds(q * 16, 16)] = neg
        return 0
    lax.fori_loop(0, NG * GP // 16, init_body, 0)

    def start(ch, gb, bb, sg, sb):
        pltpu.make_async_copy(
            g_hbm.at[pl.ds((base + ch * RC) * GP, RC * GP)], gb, sg).start()
        pltpu.make_async_copy(
            b_hbm.at[pl.ds(base + ch * RC, RC)], bb, sb).start()

    def wait(gb, bb, sg, sb):
        pltpu.make_async_copy(
            g_hbm.at[pl.ds(base * GP, RC * GP)], gb, sg).wait()
        pltpu.make_async_copy(
            b_hbm.at[pl.ds(base, RC)], bb, sb).wait()

    def flush(cur, scr):
        coff = jnp.maximum(cur, 0) * GP
        for j in range(FV):
            acc[pl.ds(coff + j * 16, 16)] = scr[j]

    def process(gb, bb, cur, scr):
        def group_body(u, carry):
            cur, scr = carry
            bv = bb[pl.ds(u * 16, 16)]
            b0 = bv[0]
            bL = bv[15]

            @pl.when(jnp.logical_or(b0 != cur, bL != cur))
            def _replay(cur=cur, scr=scr):
                flush(cur, scr)
                c, s = cur, list(scr)
                for k in range(16):
                    bs = bv[k]
                    df = jnp.full((16,), bs - c).astype(jnp.float32)
                    sf = jnp.sign(df)
                    pen = BIG * (1.0 - 2.0 * sf * sf)
                    boff = bs * GP
                    roff = (u * 16 + k) * GP
                    for j in range(FV):
                        v = jnp.maximum(gb[pl.ds(roff + j * 16, 16)],
                                        jnp.minimum(s[j], pen))
                        acc[pl.ds(boff + j * 16, 16)] = v
                        s[j] = v
                    c = bs
            c = cur
            s = list(scr)
            for k in range(16):
                bs = bv[k]
                df = jnp.full((16,), bs - c).astype(jnp.float32)
                sf = jnp.sign(df)
                pen = BIG * (1.0 - 2.0 * sf * sf)
                roff = (u * 16 + k) * GP
                for j in range(FV):
                    s[j] = jnp.maximum(gb[pl.ds(roff + j * 16, 16)],
                                       jnp.minimum(s[j], pen))
                c = bs
            return (c, tuple(s))
        return lax.fori_loop(0, RC // 16, group_body, (cur, scr))

    start(0, gb0, bb0, sg0, sb0)
    start(1, gb1, bb1, sg1, sb1)
    cur0 = jnp.int32(-1)
    scr0 = tuple(neg for _ in range(FV))

    def chunk_body(h, carry):
        cur, scr = carry
        n0 = 2 * h
        wait(gb0, bb0, sg0, sb0)
        cur, scr = process(gb0, bb0, cur, scr)

        @pl.when(n0 + 2 < NCH)
        def _():
            start(n0 + 2, gb0, bb0, sg0, sb0)

        wait(gb1, bb1, sg1, sb1)
        cur, scr = process(gb1, bb1, cur, scr)

        @pl.when(n0 + 3 < NCH)
        def _():
            start(n0 + 3, gb1, bb1, sg1, sb1)

        return (cur, scr)

    cur, scr = lax.fori_loop(0, NCH // 2, chunk_body, (cur0, scr0))
    flush(cur, scr)
    pltpu.sync_copy(acc, out_hbm.at[wid])


def _combine_body(part_ref, rW1_ref, rb1_ref, rW2_ref, rb2_ref,
                  rW3_ref, rb3_ref, out_ref):
    p = part_ref[0]
    for w in range(1, NW):
        p = jnp.maximum(p, part_ref[w])
    h = jnp.maximum(
        jnp.dot(p, rW1_ref[...], preferred_element_type=jnp.float32)
        + rb1_ref[...], 0.0)
    h = jnp.maximum(
        jnp.dot(h, rW2_ref[...], preferred_element_type=jnp.float32)
        + rb2_ref[...], 0.0)
    out_ref[...] = jnp.dot(h, rW3_ref[...],
                           preferred_element_type=jnp.float32) + rb3_ref[...]


def kernel(pos, batch, W_ffm, b_ffm, W_nnx, b_nnx, bn1_g, bn1_b, lin1_W,
           lin1_b, bn2_g, bn2_b, lin2_W, lin2_b, rc, gp_bn_g, gp_bn_b,
           gp_W, gp_b, reg_W1, reg_b1, reg_W2, reg_b2, reg_W3, reg_b3):
    inv = 1.0 / jnp.sqrt(1.0 + 1e-5)
    s1 = bn1_g * inv
    s2 = bn2_g * inv
    W_nnx_f = W_nnx * s1[0][None, :]
    b_nnx_f = (b_nnx * s1[0] + bn1_b[0]).reshape(1, H)
    lin1_Wf = lin1_W * s2[:, None, :]
    lin1_bf = lin1_b * s2 + bn2_b
    lin2_Wf = lin2_W * rc[:, None, None]
    lin2_bf = lin2_b * rc[:, None]
    gps = (gp_bn_g * inv).reshape(1, H)
    gpb2 = gp_bn_b.reshape(1, H)

    pos_pad = jnp.pad(pos, ((0, NPAD - N), (0, 0)))
    batch_pad = jnp.pad(batch, (0, NPAD - N), constant_values=NG - 1)

    full = lambda shape: pl.BlockSpec(shape, lambda i: tuple(0 for _ in shape))

    g = pl.pallas_call(
        _encoder_body,
        grid=(NB,),
        in_specs=[
            pl.BlockSpec((BLK, 3), lambda i: (i, 0)),
            full((3, H)), full((1, H)),
            full((H, H)), full((1, H)),
            full((L, H)), full((L, H)),
            full((L, H, H)), full((L, H)),
            full((L, H, H)), full((L, H)),
            full((1, H)), full((1, H)), full((H, GP)), full((1, GP)),
        ],
        out_specs=pl.BlockSpec((BLK * GP,), lambda i: (i,)),
        out_shape=jax.ShapeDtypeStruct((NPAD * GP,), jnp.float32),
    )(
        pos_pad,
        W_ffm, b_ffm.reshape(1, H),
        W_nnx_f, b_nnx_f,
        s1, bn1_b,
        lin1_Wf, lin1_bf,
        lin2_Wf, lin2_bf,
        gps, gpb2, gp_W, gp_b.reshape(1, GP),
    )

    mesh = plsc.VectorSubcoreMesh(core_axis_name="c", subcore_axis_name="s")
    partials = pl.kernel(
        _pool_body,
        out_type=jax.ShapeDtypeStruct((NW, NG * GP), jnp.float32),
        mesh=mesh,
        scratch_types=[
            pltpu.VMEM((RC * GP,), jnp.float32),
            pltpu.VMEM((RC * GP,), jnp.float32),
            pltpu.VMEM((RC,), jnp.int32),
            pltpu.VMEM((RC,), jnp.int32),
            pltpu.VMEM((NG * GP,), jnp.float32),
            pltpu.SemaphoreType.DMA,
            pltpu.SemaphoreType.DMA,
            pltpu.SemaphoreType.DMA,
            pltpu.SemaphoreType.DMA,
        ],
    )(g, batch_pad)

    out = pl.pallas_call(
        _combine_body,
        grid=(1,),
        in_specs=[
            full((NW, NG, GP)),
            full((GP, M1)), full((1, M1)),
            full((M1, M2)), full((1, M2)),
            full((M2, NC)), full((1, NC)),
        ],
        out_specs=pl.BlockSpec((NG, NC), lambda i: (0, 0)),
        out_shape=jax.ShapeDtypeStruct((NG, NC), jnp.float32),
    )(
        partials.reshape(NW, NG, GP),
        reg_W1, reg_b1.reshape(1, M1),
        reg_W2, reg_b2.reshape(1, M2),
        reg_W3, reg_b3.reshape(1, NC),
    )
    return out
